# baseline (speedup 1.0000x reference)
import functools

import jax
import jax.numpy as jnp
from jax import lax
from jax.experimental import pallas as pl
from jax.experimental.pallas import tpu as pltpu
from jax.experimental.pallas import tpu_sc as plsc

_KPAD = 128
_KPACK = 32


@functools.lru_cache(maxsize=None)
def _make_sc_gather(n_idx: int):
    info = plsc.get_sparse_core_info()
    nc, ns = info.num_cores, info.num_subcores
    nw = nc * ns
    assert n_idx % nw == 0
    b_per_w = n_idx // nw
    chunk = 128
    n_chunks = (b_per_w + chunk - 1) // chunk
    sizes = [min(chunk, b_per_w - j * chunk) for j in range(n_chunks)]
    assert all(s % 8 == 0 for s in sizes)

    mesh = plsc.VectorSubcoreMesh(core_axis_name="c", subcore_axis_name="s")

    nb = 8

    @functools.partial(
        pl.kernel,
        mesh=mesh,
        compiler_params=pltpu.CompilerParams(use_tc_tiling_on_sc=False),
        out_type=jax.ShapeDtypeStruct((n_idx, _KPAD), jnp.float32),
        scratch_types=[
            pltpu.VMEM((n_chunks, chunk), jnp.int32),
        ]
        + [pltpu.VMEM((chunk, _KPACK), jnp.float32) for _ in range(nb)]
        + [pltpu.SemaphoreType.DMA for _ in range(2 * nb)],
    )
    def gather_kernel(emb_hbm, idx_hbm, out_hbm, idx_v, *scratch):
        bufs = scratch[:nb]
        gsems = scratch[nb : 2 * nb]
        osems = scratch[2 * nb : 3 * nb]
        wid = lax.axis_index("s") * nc + lax.axis_index("c")
        base = wid * b_per_w
        pltpu.sync_copy(idx_hbm.at[wid], idx_v)
        gathers = [None] * n_chunks
        outs = [None] * n_chunks
        for k in range(min(nb - 1, n_chunks)):
            gathers[k] = pltpu.async_copy(
                emb_hbm.at[idx_v.at[k]], bufs[k % nb], gsems[k % nb]
            )
        for j in range(n_chunks):
            k = j + nb - 1
            if k < n_chunks:
                if j >= 1:
                    outs[j - 1].wait()
                gathers[k] = pltpu.async_copy(
                    emb_hbm.at[idx_v.at[k]], bufs[k % nb], gsems[k % nb]
                )
            gathers[j].wait()
            outs[j] = pltpu.async_copy(
                bufs[j % nb].at[pl.ds(0, sizes[j]), :],
                out_hbm.at[pl.ds(base + j * chunk, sizes[j]), pl.ds(0, _KPACK)],
                osems[j % nb],
            )
        for j in range(max(0, n_chunks - nb), n_chunks):
            if outs[j] is not None:
                outs[j].wait()

    def run(emb_pack, idx):
        idx2 = idx.reshape(nw, b_per_w)
        pad = n_chunks * chunk - b_per_w
        if pad:
            idx2 = jnp.pad(idx2, ((0, 0), (0, pad)))
        idx3 = idx2.reshape(nw, n_chunks, chunk)
        return gather_kernel(emb_pack, idx3)

    return run


def _mm_body(h_ref, wp_ref, b_ref, o_ref):
    for i in range(o_ref.shape[0]):
        hw = lax.bitcast_convert_type(h_ref[i, :, : _KPACK], jnp.uint32)
        ha = lax.bitcast_convert_type(hw & jnp.uint32(0xFFFF0000), jnp.float32)
        hb = lax.bitcast_convert_type(hw << jnp.uint32(16), jnp.float32)
        hcat = jnp.concatenate(
            [ha.astype(jnp.bfloat16), hb.astype(jnp.bfloat16)], axis=1
        )
        prod = lax.dot_general(
            wp_ref[...], hcat, (((1,), (1,)), ((), ())),
            preferred_element_type=jnp.float32,
        )
        o_ref[i] = prod + b_ref[...]


@functools.lru_cache(maxsize=None)
def _make_tc_head(seq: int, bsz: int, vocab: int):
    bm = 1024
    bl = 1
    assert bsz % bm == 0 and seq % bl == 0
    nt = bsz // bm
    grid = (seq // bl, nt)
    return pl.pallas_call(
        _mm_body,
        grid=grid,
        in_specs=[
            pl.BlockSpec((bl, bm, _KPAD), lambda l, t: (l, t, 0)),
            pl.BlockSpec((vocab, 2 * _KPACK), lambda l, t: (0, 0)),
            pl.BlockSpec((vocab, 1), lambda l, t: (0, 0)),
        ],
        out_specs=pl.BlockSpec((bl, vocab, bm), lambda l, t: (l, 0, t)),
        out_shape=jax.ShapeDtypeStruct((seq, vocab, bsz), jnp.float32),
    )


def kernel(x, emb, W, b):
    bsz, seq = x.shape
    vocab, emb_dim = emb.shape
    n_idx = bsz * seq
    idx = jnp.swapaxes(x, 0, 1).reshape(-1).astype(jnp.int32)
    emb_bf = lax.bitcast_convert_type(
        emb.astype(jnp.bfloat16), jnp.uint16
    ).astype(jnp.uint32)
    emb_pack = lax.bitcast_convert_type(
        (emb_bf[:, 0::2] << jnp.uint32(16)) | emb_bf[:, 1::2], jnp.float32
    )
    wp = jnp.concatenate([W[0::2].T, W[1::2].T], axis=1).astype(
        jnp.bfloat16
    )
    h = _make_sc_gather(n_idx)(emb_pack, idx)
    h3 = h.reshape(seq, bsz, _KPAD)
    out_t = _make_tc_head(seq, bsz, vocab)(h3, wp, b.reshape(vocab, 1))
    return jnp.transpose(out_t, (2, 0, 1))


# speedup vs baseline: 1.1212x; 1.1212x over previous
import functools

import jax
import jax.numpy as jnp
from jax import lax
from jax.experimental import pallas as pl
from jax.experimental.pallas import tpu as pltpu
from jax.experimental.pallas import tpu_sc as plsc

_KPAD = 128
_KPACK = 32


@functools.lru_cache(maxsize=None)
def _make_sc_gather(n_idx: int):
    info = plsc.get_sparse_core_info()
    nc, ns = info.num_cores, info.num_subcores
    nw = nc * ns
    assert n_idx % nw == 0
    b_per_w = n_idx // nw
    chunk = 80
    n_chunks = (b_per_w + chunk - 1) // chunk
    sizes = [min(chunk, b_per_w - j * chunk) for j in range(n_chunks)]
    assert all(s % 8 == 0 for s in sizes)

    mesh = plsc.VectorSubcoreMesh(core_axis_name="c", subcore_axis_name="s")

    nb = 8

    @functools.partial(
        pl.kernel,
        mesh=mesh,
        compiler_params=pltpu.CompilerParams(use_tc_tiling_on_sc=False),
        out_type=jax.ShapeDtypeStruct((n_idx, _KPAD), jnp.float32),
        scratch_types=[
            pltpu.VMEM((n_chunks, chunk), jnp.int32),
        ]
        + [pltpu.VMEM((chunk, _KPACK), jnp.float32) for _ in range(nb)]
        + [pltpu.SemaphoreType.DMA for _ in range(2 * nb)],
    )
    def gather_kernel(emb_hbm, idx_hbm, out_hbm, idx_v, *scratch):
        bufs = scratch[:nb]
        gsems = scratch[nb : 2 * nb]
        osems = scratch[2 * nb : 3 * nb]
        wid = lax.axis_index("s") * nc + lax.axis_index("c")
        base = wid * b_per_w
        pltpu.sync_copy(idx_hbm.at[wid], idx_v)
        gathers = [None] * n_chunks
        outs = [None] * n_chunks
        for k in range(min(nb - 1, n_chunks)):
            gathers[k] = pltpu.async_copy(
                emb_hbm.at[idx_v.at[k]], bufs[k % nb], gsems[k % nb]
            )
        for j in range(n_chunks):
            k = j + nb - 1
            if k < n_chunks:
                if j >= 1:
                    outs[j - 1].wait()
                gathers[k] = pltpu.async_copy(
                    emb_hbm.at[idx_v.at[k]], bufs[k % nb], gsems[k % nb]
                )
            gathers[j].wait()
            outs[j] = pltpu.async_copy(
                bufs[j % nb].at[pl.ds(0, sizes[j]), :],
                out_hbm.at[pl.ds(base + j * chunk, sizes[j]), pl.ds(0, _KPACK)],
                osems[j % nb],
            )
        for j in range(max(0, n_chunks - nb), n_chunks):
            if outs[j] is not None:
                outs[j].wait()

    def run(emb_pack, idx):
        idx2 = idx.reshape(nw, b_per_w)
        pad = n_chunks * chunk - b_per_w
        if pad:
            idx2 = jnp.pad(idx2, ((0, 0), (0, pad)))
        idx3 = idx2.reshape(nw, n_chunks, chunk)
        return gather_kernel(emb_pack, idx3)

    return run


def _mm_body(h_ref, wp_ref, b_ref, o_ref):
    for i in range(o_ref.shape[0]):
        hw = lax.bitcast_convert_type(h_ref[i, :, : _KPACK], jnp.uint32)
        ha = lax.bitcast_convert_type(hw & jnp.uint32(0xFFFF0000), jnp.float32)
        hb = lax.bitcast_convert_type(hw << jnp.uint32(16), jnp.float32)
        hcat = jnp.concatenate(
            [ha.astype(jnp.bfloat16), hb.astype(jnp.bfloat16)], axis=1
        )
        prod = lax.dot_general(
            wp_ref[...], hcat, (((1,), (1,)), ((), ())),
            preferred_element_type=jnp.float32,
        )
        o_ref[i] = prod + b_ref[...]


@functools.lru_cache(maxsize=None)
def _make_tc_head(seq: int, bsz: int, vocab: int):
    bm = 1024
    bl = 1
    assert bsz % bm == 0 and seq % bl == 0
    nt = bsz // bm
    grid = (seq // bl, nt)
    return pl.pallas_call(
        _mm_body,
        grid=grid,
        in_specs=[
            pl.BlockSpec((bl, bm, _KPAD), lambda l, t: (l, t, 0)),
            pl.BlockSpec((vocab, 2 * _KPACK), lambda l, t: (0, 0)),
            pl.BlockSpec((vocab, 1), lambda l, t: (0, 0)),
        ],
        out_specs=pl.BlockSpec((bl, vocab, bm), lambda l, t: (l, 0, t)),
        out_shape=jax.ShapeDtypeStruct((seq, vocab, bsz), jnp.float32),
    )


def kernel(x, emb, W, b):
    bsz, seq = x.shape
    vocab, emb_dim = emb.shape
    n_idx = bsz * seq
    idx = jnp.swapaxes(x, 0, 1).reshape(-1).astype(jnp.int32)
    emb_bf = lax.bitcast_convert_type(
        emb.astype(jnp.bfloat16), jnp.uint16
    ).astype(jnp.uint32)
    emb_pack = lax.bitcast_convert_type(
        (emb_bf[:, 0::2] << jnp.uint32(16)) | emb_bf[:, 1::2], jnp.float32
    )
    wp = jnp.concatenate([W[0::2].T, W[1::2].T], axis=1).astype(
        jnp.bfloat16
    )
    h = _make_sc_gather(n_idx)(emb_pack, idx)
    h3 = h.reshape(seq, bsz, _KPAD)
    out_t = _make_tc_head(seq, bsz, vocab)(h3, wp, b.reshape(vocab, 1))
    return jnp.transpose(out_t, (2, 0, 1))


# speedup vs baseline: 1.2343x; 1.1008x over previous
import functools

import jax
import jax.numpy as jnp
from jax import lax
from jax.experimental import pallas as pl
from jax.experimental.pallas import tpu as pltpu
from jax.experimental.pallas import tpu_sc as plsc

_KPAD = 128
_KPACK = 32


@functools.lru_cache(maxsize=None)
def _make_sc_gather(n_idx: int):
    info = plsc.get_sparse_core_info()
    nc, ns = info.num_cores, info.num_subcores
    nw = nc * ns
    assert n_idx % nw == 0
    b_per_w = n_idx // nw
    chunk = 80
    n_chunks = (b_per_w + chunk - 1) // chunk
    sizes = [min(chunk, b_per_w - j * chunk) for j in range(n_chunks)]
    assert all(s % 8 == 0 for s in sizes)

    mesh = plsc.VectorSubcoreMesh(core_axis_name="c", subcore_axis_name="s")

    nb = 8

    @functools.partial(
        pl.kernel,
        mesh=mesh,
        compiler_params=pltpu.CompilerParams(use_tc_tiling_on_sc=False),
        out_type=jax.ShapeDtypeStruct((n_idx, _KPAD), jnp.float32),
        scratch_types=[
            pltpu.VMEM((n_chunks, chunk), jnp.int32),
        ]
        + [pltpu.VMEM((chunk, _KPACK), jnp.float32) for _ in range(nb)]
        + [pltpu.SemaphoreType.DMA for _ in range(2 * nb)],
    )
    def gather_kernel(emb_hbm, idx_hbm, out_hbm, idx_v, *scratch):
        bufs = scratch[:nb]
        gsems = scratch[nb : 2 * nb]
        osems = scratch[2 * nb : 3 * nb]
        wid = lax.axis_index("s") * nc + lax.axis_index("c")
        base = wid * b_per_w
        pltpu.sync_copy(idx_hbm.at[wid], idx_v)
        gathers = [None] * n_chunks
        outs = [None] * n_chunks
        for k in range(min(nb - 1, n_chunks)):
            gathers[k] = pltpu.async_copy(
                emb_hbm.at[idx_v.at[k]], bufs[k % nb], gsems[k % nb]
            )
        for j in range(n_chunks):
            k = j + nb - 1
            if k < n_chunks:
                if j >= 1:
                    outs[j - 1].wait()
                gathers[k] = pltpu.async_copy(
                    emb_hbm.at[idx_v.at[k]], bufs[k % nb], gsems[k % nb]
                )
            gathers[j].wait()
            outs[j] = pltpu.async_copy(
                bufs[j % nb].at[pl.ds(0, sizes[j]), :],
                out_hbm.at[pl.ds(base + j * chunk, sizes[j]), pl.ds(0, _KPACK)],
                osems[j % nb],
            )
        for j in range(max(0, n_chunks - nb), n_chunks):
            if outs[j] is not None:
                outs[j].wait()

    def run(emb_pack, idx):
        idx2 = idx.reshape(nw, b_per_w)
        pad = n_chunks * chunk - b_per_w
        if pad:
            idx2 = jnp.pad(idx2, ((0, 0), (0, pad)))
        idx3 = idx2.reshape(nw, n_chunks, chunk)
        return gather_kernel(emb_pack, idx3)

    return run


def _mm_body(h_ref, wp_ref, b_ref, o_ref):
    for i in range(o_ref.shape[0]):
        hw = lax.bitcast_convert_type(h_ref[i, :, : _KPACK], jnp.uint32)
        ha = lax.bitcast_convert_type(hw & jnp.uint32(0xFFFF0000), jnp.float32)
        hb = lax.bitcast_convert_type(hw << jnp.uint32(16), jnp.float32)
        hcat = jnp.concatenate(
            [ha.astype(jnp.bfloat16), hb.astype(jnp.bfloat16)], axis=1
        )
        prod = lax.dot_general(
            wp_ref[...], hcat, (((1,), (1,)), ((), ())),
            preferred_element_type=jnp.float32,
        )
        o_ref[i] = prod + b_ref[...]


@functools.lru_cache(maxsize=None)
def _make_tc_head(seq: int, bsz: int, vocab: int):
    bm = 1024
    bl = 2
    assert bsz % bm == 0 and seq % bl == 0
    nt = bsz // bm
    grid = (seq // bl, nt)
    return pl.pallas_call(
        _mm_body,
        grid=grid,
        in_specs=[
            pl.BlockSpec((bl, bm, _KPAD), lambda l, t: (l, t, 0)),
            pl.BlockSpec((vocab, 2 * _KPACK), lambda l, t: (0, 0)),
            pl.BlockSpec((vocab, 1), lambda l, t: (0, 0)),
        ],
        out_specs=pl.BlockSpec((bl, vocab, bm), lambda l, t: (l, 0, t)),
        out_shape=jax.ShapeDtypeStruct((seq, vocab, bsz), jnp.float32),
    )


def kernel(x, emb, W, b):
    bsz, seq = x.shape
    vocab, emb_dim = emb.shape
    n_idx = bsz * seq
    idx = jnp.swapaxes(x, 0, 1).reshape(-1).astype(jnp.int32)
    emb_bf = lax.bitcast_convert_type(
        emb.astype(jnp.bfloat16), jnp.uint16
    ).astype(jnp.uint32)
    emb_pack = lax.bitcast_convert_type(
        (emb_bf[:, 0::2] << jnp.uint32(16)) | emb_bf[:, 1::2], jnp.float32
    )
    wp = jnp.concatenate([W[0::2].T, W[1::2].T], axis=1).astype(
        jnp.bfloat16
    )
    h = _make_sc_gather(n_idx)(emb_pack, idx)
    h3 = h.reshape(seq, bsz, _KPAD)
    out_t = _make_tc_head(seq, bsz, vocab)(h3, wp, b.reshape(vocab, 1))
    return jnp.transpose(out_t, (2, 0, 1))


# speedup vs baseline: 1.2439x; 1.0078x over previous
import functools

import jax
import jax.numpy as jnp
from jax import lax
from jax.experimental import pallas as pl
from jax.experimental.pallas import tpu as pltpu
from jax.experimental.pallas import tpu_sc as plsc

_KPAD = 128
_KPACK = 32


@functools.lru_cache(maxsize=None)
def _make_sc_gather(n_idx: int):
    info = plsc.get_sparse_core_info()
    nc, ns = info.num_cores, info.num_subcores
    nw = nc * ns
    assert n_idx % nw == 0
    b_per_w = n_idx // nw
    chunk = 80
    n_chunks = (b_per_w + chunk - 1) // chunk
    sizes = [min(chunk, b_per_w - j * chunk) for j in range(n_chunks)]
    assert all(s % 8 == 0 for s in sizes)

    mesh = plsc.VectorSubcoreMesh(core_axis_name="c", subcore_axis_name="s")

    nb = 8

    @functools.partial(
        pl.kernel,
        mesh=mesh,
        compiler_params=pltpu.CompilerParams(use_tc_tiling_on_sc=False),
        out_type=jax.ShapeDtypeStruct((n_idx, _KPAD), jnp.float32),
        scratch_types=[
            pltpu.VMEM((n_chunks, chunk), jnp.int32),
        ]
        + [pltpu.VMEM((chunk, _KPACK), jnp.float32) for _ in range(nb)]
        + [pltpu.SemaphoreType.DMA for _ in range(2 * nb)],
    )
    def gather_kernel(emb_hbm, idx_hbm, out_hbm, idx_v, *scratch):
        bufs = scratch[:nb]
        gsems = scratch[nb : 2 * nb]
        osems = scratch[2 * nb : 3 * nb]
        wid = lax.axis_index("s") * nc + lax.axis_index("c")
        base = wid * b_per_w
        pltpu.sync_copy(idx_hbm.at[wid], idx_v)
        gathers = [None] * n_chunks
        outs = [None] * n_chunks
        for k in range(min(nb - 1, n_chunks)):
            gathers[k] = pltpu.async_copy(
                emb_hbm.at[idx_v.at[k]], bufs[k % nb], gsems[k % nb]
            )
        for j in range(n_chunks):
            k = j + nb - 1
            if k < n_chunks:
                if j >= 1:
                    outs[j - 1].wait()
                gathers[k] = pltpu.async_copy(
                    emb_hbm.at[idx_v.at[k]], bufs[k % nb], gsems[k % nb]
                )
            gathers[j].wait()
            outs[j] = pltpu.async_copy(
                bufs[j % nb].at[pl.ds(0, sizes[j]), :],
                out_hbm.at[pl.ds(base + j * chunk, sizes[j]), pl.ds(0, _KPACK)],
                osems[j % nb],
            )
        for j in range(max(0, n_chunks - nb), n_chunks):
            if outs[j] is not None:
                outs[j].wait()

    def run(emb_pack, idx):
        idx2 = idx.reshape(nw, b_per_w)
        pad = n_chunks * chunk - b_per_w
        if pad:
            idx2 = jnp.pad(idx2, ((0, 0), (0, pad)))
        idx3 = idx2.reshape(nw, n_chunks, chunk)
        return gather_kernel(emb_pack, idx3)

    return run


def _mm_body(h_ref, wp_ref, b_ref, o_ref):
    for i in range(o_ref.shape[0]):
        hw = lax.bitcast_convert_type(h_ref[i, :, : _KPACK], jnp.uint32)
        ha = lax.bitcast_convert_type(hw & jnp.uint32(0xFFFF0000), jnp.float32)
        hb = lax.bitcast_convert_type(hw << jnp.uint32(16), jnp.float32)
        hcat = jnp.concatenate(
            [ha.astype(jnp.bfloat16), hb.astype(jnp.bfloat16)], axis=1
        )
        prod = lax.dot_general(
            wp_ref[...], hcat, (((1,), (1,)), ((), ())),
            preferred_element_type=jnp.float32,
        )
        o_ref[i] = prod + b_ref[...]


@functools.lru_cache(maxsize=None)
def _make_tc_head(seq: int, bsz: int, vocab: int):
    bm = 1024
    bl = 5
    assert bsz % bm == 0 and seq % bl == 0
    nt = bsz // bm
    grid = (seq // bl, nt)
    return pl.pallas_call(
        _mm_body,
        grid=grid,
        in_specs=[
            pl.BlockSpec((bl, bm, _KPAD), lambda l, t: (l, t, 0)),
            pl.BlockSpec((vocab, 2 * _KPACK), lambda l, t: (0, 0)),
            pl.BlockSpec((vocab, 1), lambda l, t: (0, 0)),
        ],
        out_specs=pl.BlockSpec((bl, vocab, bm), lambda l, t: (l, 0, t)),
        out_shape=jax.ShapeDtypeStruct((seq, vocab, bsz), jnp.float32),
    )


def kernel(x, emb, W, b):
    bsz, seq = x.shape
    vocab, emb_dim = emb.shape
    n_idx = bsz * seq
    idx = jnp.swapaxes(x, 0, 1).reshape(-1).astype(jnp.int32)
    emb_bf = lax.bitcast_convert_type(
        emb.astype(jnp.bfloat16), jnp.uint16
    ).astype(jnp.uint32)
    emb_pack = lax.bitcast_convert_type(
        (emb_bf[:, 0::2] << jnp.uint32(16)) | emb_bf[:, 1::2], jnp.float32
    )
    wp = jnp.concatenate([W[0::2].T, W[1::2].T], axis=1).astype(
        jnp.bfloat16
    )
    h = _make_sc_gather(n_idx)(emb_pack, idx)
    h3 = h.reshape(seq, bsz, _KPAD)
    out_t = _make_tc_head(seq, bsz, vocab)(h3, wp, b.reshape(vocab, 1))
    return jnp.transpose(out_t, (2, 0, 1))
